# baseline (speedup 1.0000x reference)
import jax
import jax.numpy as jnp
from jax import lax
from jax.experimental import pallas as pl
from jax.experimental.pallas import tpu as pltpu
from jax.experimental.pallas import tpu_sc as plsc

B = 16384
RANK = 32
NC = 2
NS = 16
NW = NC * NS
BPW = B // NW
L = 16
GROUPS = BPW // L


def _cpd_body(idx0_h, idx1_h, idx2_h, e0_h, e1_h, e2_h, out_h,
              idx0, idx1, idx2, rows0, rows1, rows2, out_v,
              sem0, sem1, sem2):
    wid = lax.axis_index("s") * NC + lax.axis_index("c")
    base = wid * BPW

    pltpu.sync_copy(idx0_h.at[pl.ds(base, BPW)], idx0)
    pltpu.sync_copy(idx1_h.at[pl.ds(base, BPW)], idx1)
    pltpu.sync_copy(idx2_h.at[pl.ds(base, BPW)], idx2)

    c0 = pltpu.async_copy(e0_h.at[idx0], rows0, sem0)
    c1 = pltpu.async_copy(e1_h.at[idx1], rows1, sem1)
    c2 = pltpu.async_copy(e2_h.at[idx2], rows2, sem2)
    c0.wait()
    c1.wait()
    c2.wait()

    def group(g, carry):
        row = g * L + lax.iota(jnp.int32, L)
        acc = jnp.zeros((L,), jnp.float32)
        for r in range(RANK):
            col = jnp.full((L,), r, jnp.int32)
            a = plsc.load_gather(rows0, [row, col])
            b = plsc.load_gather(rows1, [row, col])
            c = plsc.load_gather(rows2, [row, col])
            acc = acc + a * b * c
        out_v[pl.ds(g * L, L)] = acc
        return carry

    lax.fori_loop(0, GROUPS, group, 0)
    pltpu.sync_copy(out_v, out_h.at[pl.ds(base, BPW)])


def kernel(idxs, E0, E1, E2):
    idxs = idxs.astype(jnp.int32)
    idx0 = idxs[:, 0]
    idx1 = idxs[:, 1]
    idx2 = idxs[:, 2]
    mesh = plsc.VectorSubcoreMesh(core_axis_name="c", subcore_axis_name="s")
    f = pl.kernel(
        _cpd_body,
        out_type=jax.ShapeDtypeStruct((B,), jnp.float32),
        mesh=mesh,
        compiler_params=pltpu.CompilerParams(
            needs_layout_passes=False, use_tc_tiling_on_sc=False),
        scratch_types=[
            pltpu.VMEM((BPW,), jnp.int32),
            pltpu.VMEM((BPW,), jnp.int32),
            pltpu.VMEM((BPW,), jnp.int32),
            pltpu.VMEM((BPW, RANK), jnp.float32),
            pltpu.VMEM((BPW, RANK), jnp.float32),
            pltpu.VMEM((BPW, RANK), jnp.float32),
            pltpu.VMEM((BPW,), jnp.float32),
            pltpu.SemaphoreType.DMA,
            pltpu.SemaphoreType.DMA,
            pltpu.SemaphoreType.DMA,
        ],
    )
    return f(idx0, idx1, idx2, E0, E1, E2)
